# baseline (speedup 1.0000x reference)
import jax
import jax.numpy as jnp
from jax import lax
from jax.experimental import pallas as pl
from jax.experimental.pallas import tpu as pltpu
from jax.experimental.pallas import tpu_sc as plsc

N = 10000
E = 320000
D = 128
NUM_LAYERS = 4
GAMMA = 0.1
BETA = 0.5
EPSILON = 0.1

NPAD = 10240
E2 = 2 * E
NTILES = 16
CHUNK = 32
E3 = 655360
EROWS = E3 // CHUNK
SBLK = 32
QSEG = EROWS
ZROWS = 80
W3 = 3 * D
QROWS = NPAD // 4
QTILE = QROWS // NTILES
BLK = 1024
GRID = NPAD // BLK


def _agg_body(srcm_hbm, dstm_hbm, xsrc_hbm, xdst_hbm, scal_hbm, t_hbm,
              zeros_hbm, outu_hbm, outv_hbm, outw_hbm,
              scal_v, row_st, col_st, b0, b1, zbuf,
              acc, g0, g1, s0, s1):
    cid = lax.axis_index("c")
    sid = lax.axis_index("s")
    bufs = (b0, b1)
    gsems = (g0, g1)
    ssems = (s0, s1)
    outs = (outu_hbm, outv_hbm, outw_hbm)

    pltpu.sync_copy(scal_hbm, scal_v)

    for rnd in range(2):

        def do_round(q):
            sv = scal_v[...]
            cnt = sv[q]

            pltpu.sync_copy(zeros_hbm, zbuf)
            zb = sid * 3 * QTILE
            for zo in range(3 * QTILE // ZROWS):
                pltpu.sync_copy(zbuf, acc.at[pl.ds(zb + zo * ZROWS, ZROWS)])
            plsc.subcore_barrier()

            @pl.when(sid == 0)
            def _():
                pltpu.sync_copy(xsrc_hbm, row_st.at[pl.ds(0, 8)])
                pltpu.sync_copy(xdst_hbm, col_st.at[pl.ds(0, 8)])
                for xi in (2 * q, 2 * q + 1):
                    pltpu.async_copy(t_hbm.at[row_st.at[xi]], b0, g0).wait()
                    pltpu.sync_copy(b0.reshape(4 * CHUNK, D),
                                    acc.at[col_st.at[xi]], add=True)

            ct = ((cnt + NTILES * 8 - 1) // (NTILES * 8)) * 8
            my0 = q * QSEG + sid * ct
            myn = jnp.maximum(0, jnp.minimum(cnt - sid * ct, ct))
            nblk = (myn + SBLK - 1) // SBLK

            def blk_body(bi, carry):
                base_row = my0 + bi * SBLK
                pltpu.sync_copy(srcm_hbm.at[pl.ds(base_row, SBLK)], row_st)
                pltpu.sync_copy(dstm_hbm.at[pl.ds(base_row, SBLK)], col_st)
                lim = jnp.minimum(myn - bi * SBLK, SBLK)

                for b in range(2):
                    @pl.when(b < lim)
                    def _():
                        pltpu.async_copy(t_hbm.at[row_st.at[b]], bufs[b],
                                         gsems[b])

                def grp(g, c):
                    for b in range(2):
                        j = g * 2 + b

                        @pl.when(j < lim)
                        def _():
                            pltpu.make_async_copy(t_hbm.at[row_st.at[j]],
                                                  bufs[b], gsems[b]).wait()
                            pltpu.async_copy(bufs[b].reshape(4 * CHUNK, D),
                                             acc.at[col_st.at[j]],
                                             ssems[b], add=True)

                    for b in range(2):
                        j = g * 2 + b
                        jn = j + 2

                        @pl.when(j < lim)
                        def _():
                            pltpu.make_async_copy(bufs[b].reshape(4 * CHUNK,
                                                                  D),
                                                  acc.at[col_st.at[j]],
                                                  ssems[b]).wait()

                        @pl.when(jn < lim)
                        def _():
                            pltpu.async_copy(t_hbm.at[row_st.at[jn]], bufs[b],
                                             gsems[b])

                    return c

                lax.fori_loop(0, SBLK // 2, grp, 0)
                return carry

            lax.fori_loop(0, nblk, blk_body, 0)
            plsc.subcore_barrier()

            ob = q * QROWS + sid * QTILE
            for k in range(3):
                for off in (0, ZROWS):
                    pltpu.sync_copy(acc.at[pl.ds(k * QROWS + sid * QTILE
                                                 + off, ZROWS)], zbuf)
                    pltpu.sync_copy(zbuf, outs[k].at[pl.ds(ob + off, ZROWS)])
            plsc.subcore_barrier()

        @pl.when(cid == 0)
        def _():
            do_round(2 * rnd)

        @pl.when(cid == 1)
        def _():
            do_round(2 * rnd + 1)


_agg_call = pl.kernel(
    _agg_body,
    out_type=(jax.ShapeDtypeStruct((NPAD, D), jnp.float32),
              jax.ShapeDtypeStruct((NPAD, D), jnp.float32),
              jax.ShapeDtypeStruct((NPAD, D), jnp.float32)),
    mesh=plsc.VectorSubcoreMesh(core_axis_name="c", subcore_axis_name="s"),
    scratch_types=[
        pltpu.VMEM((16,), jnp.int32),
        pltpu.VMEM((SBLK, CHUNK), jnp.int32),
        pltpu.VMEM((SBLK, 4 * CHUNK), jnp.int32),
        pltpu.VMEM((CHUNK, 4, D), jnp.float32),
        pltpu.VMEM((CHUNK, 4, D), jnp.float32),
        pltpu.VMEM((ZROWS, D), jnp.float32),
        pltpu.VMEM_SHARED((3 * QROWS + CHUNK, D), jnp.float32),
        pltpu.SemaphoreType.DMA,
        pltpu.SemaphoreType.DMA,
        pltpu.SemaphoreType.DMA,
        pltpu.SemaphoreType.DMA,
    ],
)


def _embed_body(x_ref, wT_ref, b_ref, dis_ref, dinv_ref, h_ref, t_ref):
    i = pl.program_id(0)
    h = jnp.dot(x_ref[...], wT_ref[...], preferred_element_type=jnp.float32)
    h = h + b_ref[...]
    rows = i * BLK + lax.broadcasted_iota(jnp.int32, (BLK, 1), 0)
    h = jnp.where(rows < N, h, 0.0)
    h_ref[...] = h
    t_ref[:, 0, :] = dis_ref[...] * h
    t_ref[:, 1, :] = dinv_ref[...] * h
    t_ref[:, 2, :] = h
    t_ref[:, 3, :] = jnp.zeros_like(h)


def _layer_body(h_ref, u_ref, v_ref, w_ref, dis_ref, dinv_ref, q_ref,
                asWT_ref, wcT_ref, wsT_ref, b_ref, hn_ref, t_ref):
    i = pl.program_id(0)
    h = h_ref[...]
    dis = dis_ref[...]
    dinv = dinv_ref[...]
    u = u_ref[...]
    v = v_ref[...]
    w = w_ref[...]
    conv = jnp.dot(h, asWT_ref[...], preferred_element_type=jnp.float32)
    conv += jnp.dot(dis * u, wcT_ref[...], preferred_element_type=jnp.float32)
    conv += BETA * jnp.dot(v - q_ref[...] * w, wsT_ref[...],
                           preferred_element_type=jnp.float32)
    conv += b_ref[...]
    hn = h + EPSILON * jnp.tanh(conv)
    rows = i * BLK + lax.broadcasted_iota(jnp.int32, (BLK, 1), 0)
    hn = jnp.where(rows < N, hn, 0.0)
    hn_ref[...] = hn
    t_ref[:, 0, :] = dis * hn
    t_ref[:, 1, :] = dinv * hn
    t_ref[:, 2, :] = hn
    t_ref[:, 3, :] = jnp.zeros_like(hn)


def _readout_body(h_ref, wT_ref, b_ref, o_ref):
    o_ref[...] = jnp.dot(h_ref[...], wT_ref[...],
                         preferred_element_type=jnp.float32) + b_ref[...]


_vec_spec = pl.BlockSpec((BLK, 1), lambda i: (i, 0))
_mat_spec = pl.BlockSpec((BLK, D), lambda i: (i, 0))
_tab_spec = pl.BlockSpec((BLK, 4, D), lambda i: (i, 0, 0))
_w_spec = pl.BlockSpec((D, D), lambda i: (0, 0))
_b_spec = pl.BlockSpec((1, D), lambda i: (0, 0))

_h_shape = jax.ShapeDtypeStruct((NPAD, D), jnp.float32)
_t_shape = jax.ShapeDtypeStruct((NPAD, 4, D), jnp.float32)

_embed_call = pl.pallas_call(
    _embed_body,
    grid=(GRID,),
    in_specs=[_mat_spec, _w_spec, _b_spec, _vec_spec, _vec_spec],
    out_specs=[_mat_spec, _tab_spec],
    out_shape=[_h_shape, _t_shape],
)

_layer_call = pl.pallas_call(
    _layer_body,
    grid=(GRID,),
    in_specs=[_mat_spec, _mat_spec, _mat_spec, _mat_spec,
              _vec_spec, _vec_spec, _vec_spec,
              _w_spec, _w_spec, _w_spec, _b_spec],
    out_specs=[_mat_spec, _tab_spec],
    out_shape=[_h_shape, _t_shape],
)

_readout_call = pl.pallas_call(
    _readout_body,
    grid=(10,),
    in_specs=[pl.BlockSpec((1000, D), lambda i: (i, 0)), _w_spec, _b_spec],
    out_specs=pl.BlockSpec((1000, D), lambda i: (i, 0)),
    out_shape=jax.ShapeDtypeStruct((N, D), jnp.float32),
)


def kernel(x, edge_index, W_emb, b_emb, W, bias, Wc, Ws, W_out, b_out):
    row0, col0 = edge_index[0], edge_index[1]
    valid = row0 != col0
    r2 = jnp.concatenate([row0, col0])
    c2 = jnp.concatenate([col0, row0])
    v2 = jnp.concatenate([valid, valid])
    sentinel = jnp.int32(N * N)
    lin = jnp.where(v2, r2 * N + c2, sentinel)
    lin = jnp.sort(lin)
    first = jnp.concatenate([jnp.ones((1,), bool), lin[1:] != lin[:-1]])
    keep = (lin < sentinel) & first

    keep_f = keep.astype(jnp.float32)
    dstu = jnp.where(keep, lin // N, N)
    deg = jnp.zeros(N, jnp.float32).at[dstu].add(keep_f, mode="drop")
    dis = jnp.where(deg > 0, lax.rsqrt(deg), 0.0)
    dinv = jnp.where(deg > 0, 1.0 / deg, 0.0)

    cumdeg = jnp.cumsum(deg.astype(jnp.int32))
    k_tot = cumdeg[-1]
    i_n = jnp.arange(N, dtype=jnp.int32)
    rs = jnp.searchsorted(cumdeg, i_n, side="right").astype(jnp.int32)
    qv = jnp.where(i_n < k_tot, dinv[jnp.clip(rs, 0, N - 1)], 0.0)

    lin_p = jnp.concatenate([lin, jnp.full((E3 - E2,), sentinel, jnp.int32)])
    keep_p = jnp.concatenate([keep, jnp.zeros((E3 - E2,), bool)])
    src_e = jnp.where(keep_p, lin_p % N, N).astype(jnp.int32)
    dst_e = (lin_p // N).astype(jnp.int32)

    cuts = jnp.array([QROWS * N, 2 * QROWS * N, 3 * QROWS * N], jnp.int32)
    b_mid = jnp.searchsorted(lin_p, cuts, side="left").astype(jnp.int32)
    bq = jnp.concatenate([jnp.zeros((1,), jnp.int32), b_mid,
                          jnp.full((1,), E3, jnp.int32)])
    sr_q = (bq[:4] + CHUNK - 1) // CHUNK
    er_q = bq[1:] // CHUNK
    cnt_q = jnp.maximum(er_q - sr_q, 0)

    src_m = src_e.reshape(EROWS, CHUNK)
    dst_r = dst_e.reshape(EROWS, CHUNK)
    rowq = jnp.searchsorted(b_mid, jnp.arange(EROWS, dtype=jnp.int32) * CHUNK,
                            side="right").astype(jnp.int32)
    dst_m = dst_r - (QROWS * rowq)[:, None]
    koff = (jnp.arange(3, dtype=jnp.int32) * QROWS)[None, None, :]
    dump = jnp.int32(3 * QROWS) + jnp.arange(CHUNK, dtype=jnp.int32)

    xsrc = []
    xdst = []
    for q in range(4):
        c1lo = bq[q]
        c1hi = jnp.minimum(sr_q[q] * CHUNK, bq[q + 1])
        c2lo = jnp.maximum(er_q[q] * CHUNK, c1hi)
        c2hi = bq[q + 1]
        for lo, hi in ((c1lo, c1hi), (c2lo, c2hi)):
            r = jnp.clip(lo // CHUNK, 0, EROWS - 1)
            pos = r * CHUNK + jnp.arange(CHUNK, dtype=jnp.int32)
            m = (pos >= lo) & (pos < hi)
            xsrc.append(jnp.where(m, src_m[r], N))
            xdst.append(jnp.where(m, dst_r[r] - QROWS * q, 0))
    xsrc = jnp.stack(xsrc).astype(jnp.int32)
    xdst = jnp.stack(xdst).astype(jnp.int32)
    xdst = jnp.concatenate(
        [xdst[:, :, None] + koff[0][None],
         jnp.broadcast_to(dump[None, :, None], (8, CHUNK, 1))], axis=2)
    xdst = xdst.reshape(8, 4 * CHUNK)

    scal = jnp.zeros((16,), jnp.int32)
    scal = scal.at[0:4].set(cnt_q)

    pad_rows = ((0, SBLK + NTILES), (0, 0))
    src_m = jnp.pad(src_m, pad_rows, constant_values=N)
    dst_m = jnp.pad(dst_m, pad_rows, constant_values=0)
    seg_idx = (sr_q[:, None] + jnp.arange(QSEG, dtype=jnp.int32)[None, :])
    seg_idx = jnp.clip(seg_idx, 0, EROWS + SBLK + NTILES - 1).reshape(-1)
    src_m = jnp.concatenate([src_m[seg_idx],
                             jnp.full((SBLK, CHUNK), N, jnp.int32)])
    dst_m = jnp.concatenate([dst_m[seg_idx],
                             jnp.zeros((SBLK, CHUNK), jnp.int32)])
    dst_m = jnp.concatenate(
        [dst_m[:, :, None] + koff,
         jnp.broadcast_to(dump[None, :, None],
                          (dst_m.shape[0], CHUNK, 1))], axis=2)
    dst_m = dst_m.reshape(-1, 4 * CHUNK)

    pad_n = ((0, NPAD - N),)
    disp = jnp.pad(dis, pad_n)[:, None]
    dinvp = jnp.pad(dinv, pad_n)[:, None]
    qp = jnp.pad(qv, pad_n)[:, None]
    xp = jnp.pad(x, ((0, NPAD - N), (0, 0)))
    zeros_blk = jnp.zeros((ZROWS, D), jnp.float32)

    asWT = W.T - W - GAMMA * jnp.eye(D, dtype=jnp.float32)
    wt = jnp.triu(Wc, 1)
    wcT = (wt - wt.T).T
    wt = jnp.triu(Ws)
    wsT = (wt + wt.T).T
    b_emb2 = b_emb[None, :]
    bias2 = bias[None, :]
    b_out2 = b_out[None, :]

    h, t = _embed_call(xp, W_emb.T, b_emb2, disp, dinvp)
    for _ in range(NUM_LAYERS):
        u, v, w = _agg_call(src_m, dst_m, xsrc, xdst, scal, t, zeros_blk)
        h, t = _layer_call(h, u, v, w, disp, dinvp, qp,
                           asWT, wcT, wsT, bias2)
    return _readout_call(h, W_out.T, b_out2)


# speedup vs baseline: 1.1084x; 1.1084x over previous
import jax
import jax.numpy as jnp
from jax import lax
from jax.experimental import pallas as pl
from jax.experimental.pallas import tpu as pltpu
from jax.experimental.pallas import tpu_sc as plsc

N = 10000
E = 320000
D = 128
NUM_LAYERS = 4
GAMMA = 0.1
BETA = 0.5
EPSILON = 0.1

NPAD = 10240
E2 = 2 * E
NTILES = 16
CHUNK = 32
E3 = 655360
EROWS = E3 // CHUNK
SBLK = 24
QSEG = EROWS
ZROWS = 80
W3 = 3 * D
QROWS = NPAD // 4
QTILE = QROWS // NTILES
BLK = 1024
GRID = NPAD // BLK


def _agg_body(srcm_hbm, dstm_hbm, xsrc_hbm, xdst_hbm, scal_hbm, t_hbm,
              zeros_hbm, outu_hbm, outv_hbm, outw_hbm,
              scal_v, row_st, col_st, b0, b1, b2, zbuf,
              acc, g0, g1, g2, s0, s1, s2):
    cid = lax.axis_index("c")
    sid = lax.axis_index("s")
    bufs = (b0, b1, b2)
    gsems = (g0, g1, g2)
    ssems = (s0, s1, s2)
    outs = (outu_hbm, outv_hbm, outw_hbm)

    pltpu.sync_copy(scal_hbm, scal_v)

    for rnd in range(2):

        def do_round(q):
            sv = scal_v[...]
            cnt = sv[q]

            pltpu.sync_copy(zeros_hbm, zbuf)
            zb = sid * 3 * QTILE
            for zo in range(3 * QTILE // ZROWS):
                pltpu.sync_copy(zbuf, acc.at[pl.ds(zb + zo * ZROWS, ZROWS)])
            plsc.subcore_barrier()

            @pl.when(sid == 0)
            def _():
                pltpu.sync_copy(xsrc_hbm, row_st.at[pl.ds(0, 8)])
                pltpu.sync_copy(xdst_hbm, col_st.at[pl.ds(0, 8)])
                for xi in (2 * q, 2 * q + 1):
                    pltpu.async_copy(t_hbm.at[row_st.at[xi]], b0, g0).wait()
                    pltpu.sync_copy(b0.reshape(4 * CHUNK, D),
                                    acc.at[col_st.at[xi]], add=True)

            ct = ((cnt + NTILES * 8 - 1) // (NTILES * 8)) * 8
            my0 = q * QSEG + sid * ct
            myn = jnp.maximum(0, jnp.minimum(cnt - sid * ct, ct))
            nblk = (myn + SBLK - 1) // SBLK

            def blk_body(bi, carry):
                base_row = my0 + bi * SBLK
                pltpu.sync_copy(srcm_hbm.at[pl.ds(base_row, SBLK)], row_st)
                pltpu.sync_copy(dstm_hbm.at[pl.ds(base_row, SBLK)], col_st)
                lim = jnp.minimum(myn - bi * SBLK, SBLK)

                for b in range(3):
                    @pl.when(b < lim)
                    def _():
                        pltpu.async_copy(t_hbm.at[row_st.at[b]], bufs[b],
                                         gsems[b])

                def grp(g, c):
                    for b in range(3):
                        j = g * 3 + b

                        @pl.when(j < lim)
                        def _():
                            pltpu.make_async_copy(t_hbm.at[row_st.at[j]],
                                                  bufs[b], gsems[b]).wait()
                            pltpu.async_copy(bufs[b].reshape(4 * CHUNK, D),
                                             acc.at[col_st.at[j]],
                                             ssems[b], add=True)

                    for b in range(3):
                        j = g * 3 + b
                        jn = j + 3

                        @pl.when(j < lim)
                        def _():
                            pltpu.make_async_copy(bufs[b].reshape(4 * CHUNK,
                                                                  D),
                                                  acc.at[col_st.at[j]],
                                                  ssems[b]).wait()

                        @pl.when(jn < lim)
                        def _():
                            pltpu.async_copy(t_hbm.at[row_st.at[jn]], bufs[b],
                                             gsems[b])

                    return c

                lax.fori_loop(0, SBLK // 3, grp, 0)
                return carry

            lax.fori_loop(0, nblk, blk_body, 0)
            plsc.subcore_barrier()

            ob = q * QROWS + sid * QTILE
            for k in range(3):
                for off in (0, ZROWS):
                    pltpu.sync_copy(acc.at[pl.ds(k * QROWS + sid * QTILE
                                                 + off, ZROWS)], zbuf)
                    pltpu.sync_copy(zbuf, outs[k].at[pl.ds(ob + off, ZROWS)])
            plsc.subcore_barrier()

        @pl.when(cid == 0)
        def _():
            do_round(2 * rnd)

        @pl.when(cid == 1)
        def _():
            do_round(2 * rnd + 1)


_agg_call = pl.kernel(
    _agg_body,
    out_type=(jax.ShapeDtypeStruct((NPAD, D), jnp.float32),
              jax.ShapeDtypeStruct((NPAD, D), jnp.float32),
              jax.ShapeDtypeStruct((NPAD, D), jnp.float32)),
    mesh=plsc.VectorSubcoreMesh(core_axis_name="c", subcore_axis_name="s"),
    scratch_types=[
        pltpu.VMEM((16,), jnp.int32),
        pltpu.VMEM((SBLK, CHUNK), jnp.int32),
        pltpu.VMEM((SBLK, 4 * CHUNK), jnp.int32),
        pltpu.VMEM((CHUNK, 4, D), jnp.float32),
        pltpu.VMEM((CHUNK, 4, D), jnp.float32),
        pltpu.VMEM((CHUNK, 4, D), jnp.float32),
        pltpu.VMEM((ZROWS, D), jnp.float32),
        pltpu.VMEM_SHARED((3 * QROWS + 8, D), jnp.float32),
        pltpu.SemaphoreType.DMA,
        pltpu.SemaphoreType.DMA,
        pltpu.SemaphoreType.DMA,
        pltpu.SemaphoreType.DMA,
        pltpu.SemaphoreType.DMA,
        pltpu.SemaphoreType.DMA,
    ],
)


def _embed_body(x_ref, wT_ref, b_ref, dis_ref, dinv_ref, h_ref, t_ref):
    i = pl.program_id(0)
    h = jnp.dot(x_ref[...], wT_ref[...], preferred_element_type=jnp.float32)
    h = h + b_ref[...]
    rows = i * BLK + lax.broadcasted_iota(jnp.int32, (BLK, 1), 0)
    h = jnp.where(rows < N, h, 0.0)
    h_ref[...] = h
    t_ref[:, 0, :] = dis_ref[...] * h
    t_ref[:, 1, :] = dinv_ref[...] * h
    t_ref[:, 2, :] = h
    t_ref[:, 3, :] = jnp.zeros_like(h)


def _layer_body(h_ref, u_ref, v_ref, w_ref, dis_ref, dinv_ref, q_ref,
                asWT_ref, wcT_ref, wsT_ref, b_ref, hn_ref, t_ref):
    i = pl.program_id(0)
    h = h_ref[...]
    dis = dis_ref[...]
    dinv = dinv_ref[...]
    u = u_ref[...]
    v = v_ref[...]
    w = w_ref[...]
    conv = jnp.dot(h, asWT_ref[...], preferred_element_type=jnp.float32)
    conv += jnp.dot(dis * u, wcT_ref[...], preferred_element_type=jnp.float32)
    conv += BETA * jnp.dot(v - q_ref[...] * w, wsT_ref[...],
                           preferred_element_type=jnp.float32)
    conv += b_ref[...]
    hn = h + EPSILON * jnp.tanh(conv)
    rows = i * BLK + lax.broadcasted_iota(jnp.int32, (BLK, 1), 0)
    hn = jnp.where(rows < N, hn, 0.0)
    hn_ref[...] = hn
    t_ref[:, 0, :] = dis * hn
    t_ref[:, 1, :] = dinv * hn
    t_ref[:, 2, :] = hn
    t_ref[:, 3, :] = jnp.zeros_like(hn)


def _readout_body(h_ref, wT_ref, b_ref, o_ref):
    o_ref[...] = jnp.dot(h_ref[...], wT_ref[...],
                         preferred_element_type=jnp.float32) + b_ref[...]


_vec_spec = pl.BlockSpec((BLK, 1), lambda i: (i, 0))
_mat_spec = pl.BlockSpec((BLK, D), lambda i: (i, 0))
_tab_spec = pl.BlockSpec((BLK, 4, D), lambda i: (i, 0, 0))
_w_spec = pl.BlockSpec((D, D), lambda i: (0, 0))
_b_spec = pl.BlockSpec((1, D), lambda i: (0, 0))

_h_shape = jax.ShapeDtypeStruct((NPAD, D), jnp.float32)
_t_shape = jax.ShapeDtypeStruct((NPAD, 4, D), jnp.float32)

_embed_call = pl.pallas_call(
    _embed_body,
    grid=(GRID,),
    in_specs=[_mat_spec, _w_spec, _b_spec, _vec_spec, _vec_spec],
    out_specs=[_mat_spec, _tab_spec],
    out_shape=[_h_shape, _t_shape],
)

_layer_call = pl.pallas_call(
    _layer_body,
    grid=(GRID,),
    in_specs=[_mat_spec, _mat_spec, _mat_spec, _mat_spec,
              _vec_spec, _vec_spec, _vec_spec,
              _w_spec, _w_spec, _w_spec, _b_spec],
    out_specs=[_mat_spec, _tab_spec],
    out_shape=[_h_shape, _t_shape],
)

_readout_call = pl.pallas_call(
    _readout_body,
    grid=(10,),
    in_specs=[pl.BlockSpec((1000, D), lambda i: (i, 0)), _w_spec, _b_spec],
    out_specs=pl.BlockSpec((1000, D), lambda i: (i, 0)),
    out_shape=jax.ShapeDtypeStruct((N, D), jnp.float32),
)


def kernel(x, edge_index, W_emb, b_emb, W, bias, Wc, Ws, W_out, b_out):
    row0, col0 = edge_index[0], edge_index[1]
    valid = row0 != col0
    r2 = jnp.concatenate([row0, col0])
    c2 = jnp.concatenate([col0, row0])
    v2 = jnp.concatenate([valid, valid])
    sentinel = jnp.int32(N * N)
    lin = jnp.where(v2, r2 * N + c2, sentinel)
    lin = jnp.sort(lin)
    first = jnp.concatenate([jnp.ones((1,), bool), lin[1:] != lin[:-1]])
    keep = (lin < sentinel) & first

    keep_f = keep.astype(jnp.float32)
    dstu = jnp.where(keep, lin // N, N)
    deg = jnp.zeros(N, jnp.float32).at[dstu].add(keep_f, mode="drop")
    dis = jnp.where(deg > 0, lax.rsqrt(deg), 0.0)
    dinv = jnp.where(deg > 0, 1.0 / deg, 0.0)

    cumdeg = jnp.cumsum(deg.astype(jnp.int32))
    k_tot = cumdeg[-1]
    i_n = jnp.arange(N, dtype=jnp.int32)
    rs = jnp.searchsorted(cumdeg, i_n, side="right").astype(jnp.int32)
    qv = jnp.where(i_n < k_tot, dinv[jnp.clip(rs, 0, N - 1)], 0.0)

    lin_p = jnp.concatenate([lin, jnp.full((E3 - E2,), sentinel, jnp.int32)])
    keep_p = jnp.concatenate([keep, jnp.zeros((E3 - E2,), bool)])
    src_e = jnp.where(keep_p, lin_p % N, N).astype(jnp.int32)
    dst_e = (lin_p // N).astype(jnp.int32)

    cuts = jnp.array([QROWS * N, 2 * QROWS * N, 3 * QROWS * N], jnp.int32)
    b_mid = jnp.searchsorted(lin_p, cuts, side="left").astype(jnp.int32)
    bq = jnp.concatenate([jnp.zeros((1,), jnp.int32), b_mid,
                          jnp.full((1,), E3, jnp.int32)])
    sr_q = (bq[:4] + CHUNK - 1) // CHUNK
    er_q = bq[1:] // CHUNK
    cnt_q = jnp.maximum(er_q - sr_q, 0)

    src_m = src_e.reshape(EROWS, CHUNK)
    dst_r = dst_e.reshape(EROWS, CHUNK)
    rowq = jnp.searchsorted(b_mid, jnp.arange(EROWS, dtype=jnp.int32) * CHUNK,
                            side="right").astype(jnp.int32)
    dst_m = dst_r - (QROWS * rowq)[:, None]
    koff = (jnp.arange(3, dtype=jnp.int32) * QROWS)[None, None, :]
    dump = jnp.int32(3 * QROWS)

    xsrc = []
    xdst = []
    for q in range(4):
        c1lo = bq[q]
        c1hi = jnp.minimum(sr_q[q] * CHUNK, bq[q + 1])
        c2lo = jnp.maximum(er_q[q] * CHUNK, c1hi)
        c2hi = bq[q + 1]
        for lo, hi in ((c1lo, c1hi), (c2lo, c2hi)):
            r = jnp.clip(lo // CHUNK, 0, EROWS - 1)
            pos = r * CHUNK + jnp.arange(CHUNK, dtype=jnp.int32)
            m = (pos >= lo) & (pos < hi)
            xsrc.append(jnp.where(m, src_m[r], N))
            xdst.append(jnp.where(m, dst_r[r] - QROWS * q, 0))
    xsrc = jnp.stack(xsrc).astype(jnp.int32)
    xdst = jnp.stack(xdst).astype(jnp.int32)
    xdst = jnp.concatenate(
        [xdst[:, :, None] + koff[0][None],
         jnp.full((8, CHUNK, 1), dump, jnp.int32)], axis=2)
    xdst = xdst.reshape(8, 4 * CHUNK)

    scal = jnp.zeros((16,), jnp.int32)
    scal = scal.at[0:4].set(cnt_q)

    pad_rows = ((0, SBLK + NTILES), (0, 0))
    src_m = jnp.pad(src_m, pad_rows, constant_values=N)
    dst_m = jnp.pad(dst_m, pad_rows, constant_values=0)
    seg_idx = (sr_q[:, None] + jnp.arange(QSEG, dtype=jnp.int32)[None, :])
    seg_idx = jnp.clip(seg_idx, 0, EROWS + SBLK + NTILES - 1).reshape(-1)
    src_m = jnp.concatenate([src_m[seg_idx],
                             jnp.full((SBLK, CHUNK), N, jnp.int32)])
    dst_m = jnp.concatenate([dst_m[seg_idx],
                             jnp.zeros((SBLK, CHUNK), jnp.int32)])
    dst_m = jnp.concatenate(
        [dst_m[:, :, None] + koff,
         jnp.full((dst_m.shape[0], CHUNK, 1), dump, jnp.int32)], axis=2)
    dst_m = dst_m.reshape(-1, 4 * CHUNK)

    pad_n = ((0, NPAD - N),)
    disp = jnp.pad(dis, pad_n)[:, None]
    dinvp = jnp.pad(dinv, pad_n)[:, None]
    qp = jnp.pad(qv, pad_n)[:, None]
    xp = jnp.pad(x, ((0, NPAD - N), (0, 0)))
    zeros_blk = jnp.zeros((ZROWS, D), jnp.float32)

    asWT = W.T - W - GAMMA * jnp.eye(D, dtype=jnp.float32)
    wt = jnp.triu(Wc, 1)
    wcT = (wt - wt.T).T
    wt = jnp.triu(Ws)
    wsT = (wt + wt.T).T
    b_emb2 = b_emb[None, :]
    bias2 = bias[None, :]
    b_out2 = b_out[None, :]

    h, t = _embed_call(xp, W_emb.T, b_emb2, disp, dinvp)
    for _ in range(NUM_LAYERS):
        u, v, w = _agg_call(src_m, dst_m, xsrc, xdst, scal, t, zeros_blk)
        h, t = _layer_call(h, u, v, w, disp, dinvp, qp,
                           asWT, wcT, wsT, bias2)
    return _readout_call(h, W_out.T, b_out2)
